# baseline (speedup 1.0000x reference)
import jax
import jax.numpy as jnp
from jax import lax
from jax.experimental import pallas as pl
from jax.experimental.pallas import tpu as pltpu
from jax.experimental.pallas import tpu_sc as plsc

_VOCAB = 1000000
_MAXLEN = 200
_DIM = 64
_BATCH = 4096

_NC, _NS = 2, 16
_NW = _NC * _NS
_BPW = _BATCH // _NW


def _body(xt_hbm, tok_hbm, pos_hbm, q_hbm,
          xv, pos_v, r0, r1, o0, o1, g0, g1, w0, w1):
    wid = lax.axis_index("s") * _NC + lax.axis_index("c")
    b0 = wid * _BPW
    rows = (r0, r1)
    outs = (o0, o1)
    gsems = (g0, g1)
    wsems = (w0, w1)

    pltpu.sync_copy(xt_hbm.at[:, pl.ds(b0, _BPW)], xv)
    pltpu.sync_copy(pos_hbm, pos_v)

    def issue_gather(s, b):
        pltpu.async_copy(tok_hbm.at[xv.at[s]], rows[b], gsems[b])

    def wait_gather(s, b):
        pltpu.make_async_copy(tok_hbm.at[xv.at[s]], rows[b], gsems[b]).wait()

    def issue_write(s, b):
        pltpu.async_copy(outs[b], q_hbm.at[s, :, wid, :, :], wsems[b])

    def wait_write(s, b):
        pltpu.make_async_copy(outs[b], q_hbm.at[s, :, wid, :, :],
                              wsems[b]).wait()

    lane = lax.iota(jnp.int32, 16)
    issue_gather(0, 0)

    def s_body(s, carry):
        for b in range(2):
            sb = s * 2 + b
            pl.when(sb + 1 < _MAXLEN)(lambda: issue_gather(sb + 1, 1 - b))
            wait_gather(sb, b)
            pl.when(sb >= 2)(lambda: wait_write(sb - 2, b))

            pvs = [pos_v[sb, pl.ds(k * 16, 16)] for k in range(_DIM // 16)]
            i0s = [(16 * k + lane) >> 3 for k in range(_DIM // 16)]
            i1 = lane & 7

            def t_body(t, c2):
                i2 = jnp.full((16,), t, jnp.int32)
                for k in range(_DIM // 16):
                    v = rows[b][t, pl.ds(k * 16, 16)] + pvs[k]
                    plsc.store_scatter(outs[b], [i0s[k], i1, i2], v)
                return c2

            lax.fori_loop(0, _BPW, t_body, 0)
            issue_write(sb, b)
        return carry

    lax.fori_loop(0, _MAXLEN // 2, s_body, 0)
    wait_write(_MAXLEN - 2, 0)
    wait_write(_MAXLEN - 1, 1)


@jax.jit
def _embed(xt, token_table, pos_table):
    mesh = plsc.VectorSubcoreMesh(core_axis_name="c", subcore_axis_name="s")
    run = pl.kernel(
        _body,
        out_type=jax.ShapeDtypeStruct(
            (_MAXLEN, _DIM // 8, _NW, 8, _BPW), jnp.float32),
        mesh=mesh,
        scratch_types=(
            [pltpu.VMEM((_MAXLEN, _BPW), jnp.int32),
             pltpu.VMEM((_MAXLEN, _DIM), jnp.float32)]
            + [pltpu.VMEM((_BPW, _DIM), jnp.float32)] * 2
            + [pltpu.VMEM((_DIM // 8, 8, _BPW), jnp.float32)] * 2
            + [pltpu.SemaphoreType.DMA] * 4
        ),
        compiler_params=pltpu.CompilerParams(
            use_tc_tiling_on_sc=False, needs_layout_passes=False),
    )
    return run(xt, token_table, pos_table)


def kernel(x, token_table, pos_table):
    xt = x.T.astype(jnp.int32)
    q = _embed(xt, token_table, pos_table)
    return q.transpose(2, 4, 0, 1, 3).reshape(_BATCH, _MAXLEN, _DIM)


# speedup vs baseline: 1.5400x; 1.5400x over previous
import jax
import jax.numpy as jnp
from jax import lax
from jax.experimental import pallas as pl
from jax.experimental.pallas import tpu as pltpu
from jax.experimental.pallas import tpu_sc as plsc

_VOCAB = 1000000
_MAXLEN = 200
_DIM = 64
_BATCH = 4096

_NC, _NS = 2, 16
_NW = _NC * _NS
_BPW = _BATCH // _NW


def _body(xt_hbm, tok_hbm, pos_hbm, q_hbm,
          xv, pos_v, r0, r1, o0, o1, g0, g1, w0, w1):
    wid = lax.axis_index("s") * _NC + lax.axis_index("c")
    b0 = wid * _BPW
    rows = (r0, r1)
    outs = (o0, o1)
    gsems = (g0, g1)
    wsems = (w0, w1)

    pltpu.sync_copy(xt_hbm.at[:, pl.ds(b0, _BPW)], xv)
    pltpu.sync_copy(pos_hbm, pos_v)

    def issue_gather(s, b):
        pltpu.async_copy(tok_hbm.at[xv.at[s]], rows[b], gsems[b])

    def wait_gather(s, b):
        pltpu.make_async_copy(tok_hbm.at[xv.at[s]], rows[b], gsems[b]).wait()

    def issue_write(s, b):
        pltpu.async_copy(outs[b], q_hbm.at[s, :, wid, :, :], wsems[b])

    def wait_write(s, b):
        pltpu.make_async_copy(outs[b], q_hbm.at[s, :, wid, :, :],
                              wsems[b]).wait()

    lane = lax.iota(jnp.int32, 16)
    issue_gather(0, 0)

    def s_body(s, carry):
        for b in range(2):
            sb = s * 2 + b
            pl.when(sb + 1 < _MAXLEN)(lambda: issue_gather(sb + 1, 1 - b))
            wait_gather(sb, b)
            pl.when(sb >= 2)(lambda: wait_write(sb - 2, b))

            pvs = [pos_v[sb, pl.ds(k * 16, 16)] for k in range(_DIM // 16)]
            cols = [16 * k + lane for k in range(_DIM // 16)]
            i0s = [(16 * k + lane) >> 3 for k in range(_DIM // 16)]
            i1 = lane & 7

            def j_body(j, c2):
                rowmix = (lane + j) & 15
                for g in range(_BPW // 16):
                    trow = rowmix + 16 * g
                    for k in range(_DIM // 16):
                        v = plsc.load_gather(rows[b], [trow, cols[k]]) + pvs[k]
                        plsc.store_scatter(outs[b], [i0s[k], i1, trow], v)
                return c2

            lax.fori_loop(0, 16, j_body, 0)
            issue_write(sb, b)
        return carry

    lax.fori_loop(0, _MAXLEN // 2, s_body, 0)
    wait_write(_MAXLEN - 2, 0)
    wait_write(_MAXLEN - 1, 1)


@jax.jit
def _embed(xt, token_table, pos_table):
    mesh = plsc.VectorSubcoreMesh(core_axis_name="c", subcore_axis_name="s")
    run = pl.kernel(
        _body,
        out_type=jax.ShapeDtypeStruct(
            (_MAXLEN, _DIM // 8, _NW, 8, _BPW), jnp.float32),
        mesh=mesh,
        scratch_types=(
            [pltpu.VMEM((_MAXLEN, _BPW), jnp.int32),
             pltpu.VMEM((_MAXLEN, _DIM), jnp.float32)]
            + [pltpu.VMEM((_BPW, _DIM), jnp.float32)] * 2
            + [pltpu.VMEM((_DIM // 8, 8, _BPW), jnp.float32)] * 2
            + [pltpu.SemaphoreType.DMA] * 4
        ),
        compiler_params=pltpu.CompilerParams(
            use_tc_tiling_on_sc=False, needs_layout_passes=False),
    )
    return run(xt, token_table, pos_table)


def kernel(x, token_table, pos_table):
    xt = x.T.astype(jnp.int32)
    q = _embed(xt, token_table, pos_table)
    return q.transpose(2, 4, 0, 1, 3).reshape(_BATCH, _MAXLEN, _DIM)


# speedup vs baseline: 2.1835x; 1.4179x over previous
import jax
import jax.numpy as jnp
from jax import lax
from jax.experimental import pallas as pl
from jax.experimental.pallas import tpu as pltpu
from jax.experimental.pallas import tpu_sc as plsc

_VOCAB = 1000000
_MAXLEN = 200
_DIM = 64
_BATCH = 4096

_NC, _NS = 2, 16
_NW = _NC * _NS
_BPW = _BATCH // _NW


def _body(xt_hbm, tok_hbm, pos_hbm, q_hbm,
          xv, pos_v, r0, r1, o0, o1, g0, g1, w0, w1):
    wid = lax.axis_index("s") * _NC + lax.axis_index("c")
    b0 = wid * _BPW
    rows = (r0, r1)
    outs = (o0, o1)
    gsems = (g0, g1)
    wsems = (w0, w1)

    pltpu.sync_copy(xt_hbm.at[:, pl.ds(b0, _BPW)], xv)
    pltpu.sync_copy(pos_hbm, pos_v)

    def issue_gather(s, b):
        pltpu.async_copy(tok_hbm.at[xv.at[s]], rows[b], gsems[b])

    def wait_gather(s, b):
        pltpu.make_async_copy(tok_hbm.at[xv.at[s]], rows[b], gsems[b]).wait()

    def issue_write(s, b):
        pltpu.async_copy(outs[b], q_hbm.at[s, :, wid, :, :], wsems[b])

    def wait_write(s, b):
        pltpu.make_async_copy(outs[b], q_hbm.at[s, :, wid, :, :],
                              wsems[b]).wait()

    lane = lax.iota(jnp.int32, 16)
    issue_gather(0, 0)

    def s_body(s, carry):
        for b in range(2):
            sb = s * 2 + b
            pl.when(sb + 1 < _MAXLEN)(lambda: issue_gather(sb + 1, 1 - b))
            wait_gather(sb, b)
            pl.when(sb >= 2)(lambda: wait_write(sb - 2, b))

            pvs = [pos_v[sb, pl.ds(k * 16, 16)] for k in range(_DIM // 16)]
            cols = [16 * k + lane for k in range(_DIM // 16)]
            i0s = [(16 * k + lane) >> 3 for k in range(_DIM // 16)]
            i1 = lane & 7

            def j_body(j, c2):
                rowmix = (lane + j) & 15
                trows = [rowmix + 16 * g for g in range(_BPW // 16)]
                for ks in ((0, 1), (2, 3)):
                    vals = []
                    for g in range(_BPW // 16):
                        for k in ks:
                            vals.append(
                                (g, k,
                                 plsc.load_gather(rows[b], [trows[g], cols[k]])
                                 + pvs[k]))
                    for g, k, v in vals:
                        plsc.store_scatter(outs[b], [i0s[k], i1, trows[g]], v)
                return c2

            lax.fori_loop(0, 16, j_body, 0)
            issue_write(sb, b)
        return carry

    lax.fori_loop(0, _MAXLEN // 2, s_body, 0)
    wait_write(_MAXLEN - 2, 0)
    wait_write(_MAXLEN - 1, 1)


@jax.jit
def _embed(xt, token_table, pos_table):
    mesh = plsc.VectorSubcoreMesh(core_axis_name="c", subcore_axis_name="s")
    run = pl.kernel(
        _body,
        out_type=jax.ShapeDtypeStruct(
            (_MAXLEN, _DIM // 8, _NW, 8, _BPW), jnp.float32),
        mesh=mesh,
        scratch_types=(
            [pltpu.VMEM((_MAXLEN, _BPW), jnp.int32),
             pltpu.VMEM((_MAXLEN, _DIM), jnp.float32)]
            + [pltpu.VMEM((_BPW, _DIM), jnp.float32)] * 2
            + [pltpu.VMEM((_DIM // 8, 8, _BPW), jnp.float32)] * 2
            + [pltpu.SemaphoreType.DMA] * 4
        ),
        compiler_params=pltpu.CompilerParams(
            use_tc_tiling_on_sc=False, needs_layout_passes=False),
    )
    return run(xt, token_table, pos_table)


def kernel(x, token_table, pos_table):
    xt = x.T.astype(jnp.int32)
    q = _embed(xt, token_table, pos_table)
    return q.transpose(2, 4, 0, 1, 3).reshape(_BATCH, _MAXLEN, _DIM)
